# baseline (speedup 1.0000x reference)
import functools

import jax
import jax.numpy as jnp
from jax import lax
from jax.experimental import pallas as pl
from jax.experimental.pallas import tpu as pltpu
from jax.experimental.pallas import tpu_sc as plsc

_NC = 2
_NSPLIT = 4


def _copy_body(rows_per_core, emb, out, *sems):
    cid = lax.axis_index("c")
    base = cid * rows_per_core
    slab = rows_per_core // _NSPLIT
    cps = []
    for i in range(_NSPLIT):
        cps.append(
            pltpu.async_copy(
                emb.at[pl.ds(base + i * slab, slab)],
                out.at[pl.ds(base + i * slab, slab)],
                sems[i],
            )
        )
    for cp in cps:
        cp.wait()


def kernel(x, emb):
    seq = x.shape[1]
    dim = emb.shape[1]
    rows_per_core = seq // _NC
    mesh = plsc.ScalarSubcoreMesh(axis_name="c", num_cores=_NC)
    run = pl.kernel(
        functools.partial(_copy_body, rows_per_core),
        out_type=jax.ShapeDtypeStruct((seq, dim), emb.dtype),
        mesh=mesh,
        scratch_types=[pltpu.SemaphoreType.DMA for _ in range(_NSPLIT)],
        compiler_params=pltpu.CompilerParams(skip_device_barrier=True),
    )
    return run(emb)


# speedup vs baseline: 23.2177x; 23.2177x over previous
import functools

import jax
import jax.numpy as jnp
from jax import lax
from jax.experimental import pallas as pl
from jax.experimental.pallas import tpu as pltpu
from jax.experimental.pallas import tpu_sc as plsc

_NC = 2
_SLAB = 512
_NBUF = 3


def _copy_body(n_slabs, emb, out, *refs):
    bufs = refs[:_NBUF]
    lsems = refs[_NBUF:2 * _NBUF]
    ssems = refs[2 * _NBUF:]
    cid = lax.axis_index("c")
    base = cid * (n_slabs * _SLAB)

    cps_l = [None] * _NBUF
    cps_s = [None] * _NBUF
    for c in range(min(_NBUF - 1, n_slabs)):
        cps_l[c] = pltpu.async_copy(
            emb.at[pl.ds(base + c * _SLAB, _SLAB)], bufs[c], lsems[c]
        )
    for c in range(n_slabs):
        i = c % _NBUF
        f = c + _NBUF - 1
        if f < n_slabs:
            j = f % _NBUF
            if cps_s[j] is not None:
                cps_s[j].wait()
                cps_s[j] = None
            cps_l[j] = pltpu.async_copy(
                emb.at[pl.ds(base + f * _SLAB, _SLAB)], bufs[j], lsems[j]
            )
        cps_l[i].wait()
        cps_s[i] = pltpu.async_copy(
            bufs[i], out.at[pl.ds(base + c * _SLAB, _SLAB)], ssems[i]
        )
    for i in range(_NBUF):
        if cps_s[i] is not None:
            cps_s[i].wait()


def kernel(x, emb):
    seq = x.shape[1]
    dim = emb.shape[1]
    n_slabs = seq // (_NC * _SLAB)
    mesh = plsc.ScalarSubcoreMesh(axis_name="c", num_cores=_NC)
    run = pl.kernel(
        functools.partial(_copy_body, n_slabs),
        out_type=jax.ShapeDtypeStruct((seq, dim), emb.dtype),
        mesh=mesh,
        scratch_types=(
            [pltpu.VMEM_SHARED((_SLAB, dim), emb.dtype) for _ in range(_NBUF)]
            + [pltpu.SemaphoreType.DMA for _ in range(2 * _NBUF)]
        ),
        compiler_params=pltpu.CompilerParams(skip_device_barrier=True),
    )
    return run(emb)


# speedup vs baseline: 23.4299x; 1.0091x over previous
import functools

import jax
import jax.numpy as jnp
from jax import lax
from jax.experimental import pallas as pl
from jax.experimental.pallas import tpu as pltpu
from jax.experimental.pallas import tpu_sc as plsc

_NC = 2
_NS = 16
_NW = _NC * _NS

_CHUNK = 32
_NBUF = 3
_SC_ROWS = 4096
_TC_BLOCK = 512


def _sc_body(n_chunks, emb, out, *refs):
    bufs = refs[:_NBUF]
    lsems = refs[_NBUF:2 * _NBUF]
    ssems = refs[2 * _NBUF:]
    wid = lax.axis_index("s") * _NC + lax.axis_index("c")
    base = wid * (n_chunks * _CHUNK)

    cps_l = [None] * _NBUF
    cps_s = [None] * _NBUF
    for c in range(min(_NBUF - 1, n_chunks)):
        cps_l[c] = pltpu.async_copy(
            emb.at[pl.ds(base + c * _CHUNK, _CHUNK)], bufs[c], lsems[c]
        )
    for c in range(n_chunks):
        i = c % _NBUF
        f = c + _NBUF - 1
        if f < n_chunks:
            j = f % _NBUF
            if cps_s[j] is not None:
                cps_s[j].wait()
                cps_s[j] = None
            cps_l[j] = pltpu.async_copy(
                emb.at[pl.ds(base + f * _CHUNK, _CHUNK)], bufs[j], lsems[j]
            )
        cps_l[i].wait()
        cps_s[i] = pltpu.async_copy(
            bufs[i], out.at[pl.ds(base + c * _CHUNK, _CHUNK)], ssems[i]
        )
    for i in range(_NBUF):
        if cps_s[i] is not None:
            cps_s[i].wait()


def _tc_body(emb_ref, part_ref, out_ref):
    del part_ref
    out_ref[...] = emb_ref[...]


def kernel(x, emb):
    seq = x.shape[1]
    dim = emb.shape[1]
    n_chunks = _SC_ROWS // (_NW * _CHUNK)
    mesh = plsc.VectorSubcoreMesh(core_axis_name="c", subcore_axis_name="s")
    sc_run = pl.kernel(
        functools.partial(_sc_body, n_chunks),
        out_type=jax.ShapeDtypeStruct((seq, dim), emb.dtype),
        mesh=mesh,
        scratch_types=(
            [pltpu.VMEM((_CHUNK, dim), emb.dtype) for _ in range(_NBUF)]
            + [pltpu.SemaphoreType.DMA for _ in range(2 * _NBUF)]
        ),
        compiler_params=pltpu.CompilerParams(skip_device_barrier=True),
    )
    part = sc_run(emb)

    base_blk = _SC_ROWS // _TC_BLOCK
    grid = (seq - _SC_ROWS) // _TC_BLOCK
    out = pl.pallas_call(
        _tc_body,
        grid=(grid,),
        in_specs=[
            pl.BlockSpec((_TC_BLOCK, dim), lambda g: (base_blk + g, 0)),
            pl.BlockSpec(memory_space=pltpu.MemorySpace.HBM),
        ],
        out_specs=pl.BlockSpec((_TC_BLOCK, dim), lambda g: (base_blk + g, 0)),
        out_shape=jax.ShapeDtypeStruct((seq, dim), emb.dtype),
        input_output_aliases={1: 0},
    )(emb, part)
    return out


# speedup vs baseline: 24.5161x; 1.0464x over previous
import functools

import jax
import jax.numpy as jnp
from jax import lax
from jax.experimental import pallas as pl
from jax.experimental.pallas import tpu as pltpu
from jax.experimental.pallas import tpu_sc as plsc

_NC = 2
_NS = 16
_NW = _NC * _NS

_CHUNK = 32
_NBUF = 3
_SC_ROWS = 4096
_TC_BLOCK = 512


def _sc_ring_copy(n_chunks, emb, out, *refs):
    bufs = refs[:_NBUF]
    lsems = refs[_NBUF:2 * _NBUF]
    ssems = refs[2 * _NBUF:]
    wid = lax.axis_index("s") * _NC + lax.axis_index("c")
    base = wid * (n_chunks * _CHUNK)

    cps_l = [None] * _NBUF
    cps_s = [None] * _NBUF
    for c in range(min(_NBUF - 1, n_chunks)):
        cps_l[c] = pltpu.async_copy(
            emb.at[pl.ds(base + c * _CHUNK, _CHUNK)], bufs[c], lsems[c]
        )
    for c in range(n_chunks):
        i = c % _NBUF
        f = c + _NBUF - 1
        if f < n_chunks:
            j = f % _NBUF
            if cps_s[j] is not None:
                cps_s[j].wait()
                cps_s[j] = None
            cps_l[j] = pltpu.async_copy(
                emb.at[pl.ds(base + f * _CHUNK, _CHUNK)], bufs[j], lsems[j]
            )
        cps_l[i].wait()
        cps_s[i] = pltpu.async_copy(
            bufs[i], out.at[pl.ds(base + c * _CHUNK, _CHUNK)], ssems[i]
        )
    for i in range(_NBUF):
        if cps_s[i] is not None:
            cps_s[i].wait()


def _tc_body(emb_ref, out_ref):
    out_ref[...] = emb_ref[...]


def kernel(x, emb):
    seq = x.shape[1]
    dim = emb.shape[1]

    base_blk = _SC_ROWS // _TC_BLOCK
    grid = (seq - _SC_ROWS) // _TC_BLOCK
    part = pl.pallas_call(
        _tc_body,
        grid=(grid,),
        in_specs=[pl.BlockSpec((_TC_BLOCK, dim), lambda g: (base_blk + g, 0))],
        out_specs=pl.BlockSpec((_TC_BLOCK, dim), lambda g: (base_blk + g, 0)),
        out_shape=jax.ShapeDtypeStruct((seq, dim), emb.dtype),
    )(emb)

    n_chunks = _SC_ROWS // (_NW * _CHUNK)
    mesh = plsc.VectorSubcoreMesh(core_axis_name="c", subcore_axis_name="s")

    def stateful(refs):
        emb_ref, out_ref = refs

        @pl.core_map(
            mesh,
            compiler_params=pltpu.CompilerParams(skip_device_barrier=True),
            scratch_shapes=(
                [pltpu.VMEM((_CHUNK, dim), emb.dtype) for _ in range(_NBUF)]
                + [pltpu.SemaphoreType.DMA for _ in range(2 * _NBUF)]
            ),
        )
        def _sc(*scratch):
            _sc_ring_copy(n_chunks, emb_ref, out_ref, *scratch)

    _, out = pl.run_state(stateful)((emb, part))
    return out


# speedup vs baseline: 24.7442x; 1.0093x over previous
import functools

import jax
import jax.numpy as jnp
from jax import lax
from jax.experimental import pallas as pl
from jax.experimental.pallas import tpu as pltpu
from jax.experimental.pallas import tpu_sc as plsc

_NC = 2
_NS = 16
_NW = _NC * _NS

_CHUNK = 16
_NBUF = 6


def _copy_body(n_chunks, emb, out, *refs):
    bufs = refs[:_NBUF]
    lsems = refs[_NBUF:2 * _NBUF]
    ssems = refs[2 * _NBUF:]
    wid = lax.axis_index("s") * _NC + lax.axis_index("c")
    base = wid * (n_chunks * _CHUNK)

    cps_l = [None] * _NBUF
    cps_s = [None] * _NBUF
    for c in range(min(_NBUF - 1, n_chunks)):
        cps_l[c] = pltpu.async_copy(
            emb.at[pl.ds(base + c * _CHUNK, _CHUNK)], bufs[c], lsems[c]
        )
    for c in range(n_chunks):
        i = c % _NBUF
        f = c + _NBUF - 1
        if f < n_chunks:
            j = f % _NBUF
            if cps_s[j] is not None:
                cps_s[j].wait()
                cps_s[j] = None
            cps_l[j] = pltpu.async_copy(
                emb.at[pl.ds(base + f * _CHUNK, _CHUNK)], bufs[j], lsems[j]
            )
        cps_l[i].wait()
        cps_s[i] = pltpu.async_copy(
            bufs[i], out.at[pl.ds(base + c * _CHUNK, _CHUNK)], ssems[i]
        )
    for i in range(_NBUF):
        if cps_s[i] is not None:
            cps_s[i].wait()


def kernel(x, emb):
    seq = x.shape[1]
    dim = emb.shape[1]
    n_chunks = seq // (_NW * _CHUNK)
    mesh = plsc.VectorSubcoreMesh(core_axis_name="c", subcore_axis_name="s")
    run = pl.kernel(
        functools.partial(_copy_body, n_chunks),
        out_type=jax.ShapeDtypeStruct((seq, dim), emb.dtype),
        mesh=mesh,
        scratch_types=(
            [pltpu.VMEM((_CHUNK, dim), emb.dtype) for _ in range(_NBUF)]
            + [pltpu.SemaphoreType.DMA for _ in range(2 * _NBUF)]
        ),
        compiler_params=pltpu.CompilerParams(skip_device_barrier=True),
    )
    return run(emb)


# speedup vs baseline: 24.9047x; 1.0065x over previous
import functools

import jax
from jax import lax
from jax.experimental import pallas as pl
from jax.experimental.pallas import tpu as pltpu
from jax.experimental.pallas import tpu_sc as plsc

_NC = 2
_NS = 16
_NW = _NC * _NS

_CHUNK = 16
_NBUF = 6


def _copy_body(n_chunks, emb, out, *refs):
    bufs = refs[:_NBUF]
    lsems = refs[_NBUF:2 * _NBUF]
    ssems = refs[2 * _NBUF:]
    wid = lax.axis_index("s") * _NC + lax.axis_index("c")
    base = wid * (n_chunks * _CHUNK)

    cps_l = [None] * _NBUF
    cps_s = [None] * _NBUF
    for c in range(min(_NBUF - 1, n_chunks)):
        cps_l[c] = pltpu.async_copy(
            emb.at[pl.ds(base + c * _CHUNK, _CHUNK)], bufs[c], lsems[c]
        )
    for c in range(n_chunks):
        i = c % _NBUF
        f = c + _NBUF - 1
        if f < n_chunks:
            j = f % _NBUF
            if cps_s[j] is not None:
                cps_s[j].wait()
                cps_s[j] = None
            cps_l[j] = pltpu.async_copy(
                emb.at[pl.ds(base + f * _CHUNK, _CHUNK)], bufs[j], lsems[j]
            )
        cps_l[i].wait()
        cps_s[i] = pltpu.async_copy(
            bufs[i], out.at[pl.ds(base + c * _CHUNK, _CHUNK)], ssems[i]
        )
    for i in range(_NBUF):
        if cps_s[i] is not None:
            cps_s[i].wait()


def kernel(x, emb):
    seq = x.shape[1]
    dim = emb.shape[1]
    n_chunks = seq // (_NW * _CHUNK)
    mesh = plsc.VectorSubcoreMesh(core_axis_name="c", subcore_axis_name="s")
    run = pl.kernel(
        functools.partial(_copy_body, n_chunks),
        out_type=jax.ShapeDtypeStruct((seq, dim), emb.dtype),
        mesh=mesh,
        scratch_types=(
            [pltpu.VMEM((_CHUNK, dim), emb.dtype) for _ in range(_NBUF)]
            + [pltpu.SemaphoreType.DMA for _ in range(2 * _NBUF)]
        ),
    )
    return run(emb)
